# baseline (speedup 1.0000x reference)
import functools

import jax
import jax.numpy as jnp
from jax import lax
from jax.experimental import pallas as pl
from jax.experimental.pallas import tpu as pltpu
from jax.experimental.pallas import tpu_sc as plsc

VOCAB = 100000
BATCH = 16384
DIM = 300

NC = 2
NS = 16
NW = NC * NS
CHUNK = 64
ROWS_PER_W = BATCH // NW
N_CHUNKS = ROWS_PER_W // CHUNK
SUBW = 16
NSUB = 20
NROWS16 = VOCAB * DIM // SUBW
MAXSUB = NROWS16 - 1

_mesh = plsc.VectorSubcoreMesh(core_axis_name="c", subcore_axis_name="s")


@functools.partial(
    pl.kernel,
    mesh=_mesh,
    out_type=jax.ShapeDtypeStruct((BATCH, DIM), jnp.float32),
    scratch_types=[
        pltpu.VMEM((ROWS_PER_W,), jnp.int32),
        pltpu.VMEM((N_CHUNKS, CHUNK), jnp.int32),
        pltpu.VMEM((N_CHUNKS, NSUB, CHUNK), jnp.int32),
        pltpu.VMEM((2, NSUB, CHUNK, SUBW), jnp.float32),
        pltpu.VMEM((2, CHUNK, DIM), jnp.float32),
        pltpu.SemaphoreType.DMA,
        pltpu.SemaphoreType.DMA,
    ],
    compiler_params=pltpu.CompilerParams(
        use_tc_tiling_on_sc=False, needs_layout_passes=False),
)
def _gather_kernel(tab16, poses_hbm, out_hbm, idx_v, hv, sidx, stage, buf,
                   gsem, wsem):
    wid = lax.axis_index("s") * NC + lax.axis_index("c")
    base = wid * ROWS_PER_W
    pltpu.sync_copy(poses_hbm.at[pl.ds(base, ROWS_PER_W)], idx_v)
    iota = lax.broadcasted_iota(jnp.int32, (16,), 0)

    for c in range(N_CHUNKS):
        for t in range(CHUNK // 16):
            ids = idx_v[pl.ds(c * CHUNK + t * 16, 16)]
            w0 = ids * DIM
            s0 = lax.shift_right_logical(w0, 4)
            hv[c, pl.ds(t * 16, 16)] = jnp.bitwise_and(w0, 15)
            for g in range(NSUB):
                sidx[c, g, pl.ds(t * 16, 16)] = jnp.minimum(s0 + g, MAXSUB)

    def gathers(c, b):
        return [
            pltpu.async_copy(tab16.at[sidx.at[c, g]], stage.at[b, g], gsem)
            for g in range(NSUB)
        ]

    def realign(c, b):
        def body(r, carry):
            rs = jnp.full((16,), r, jnp.int32)
            hs = plsc.load_gather(hv.at[c], [rs])
            t = hs + iota
            a = lax.shift_right_logical(t, 4)
            w = jnp.bitwise_and(t, 15)
            for v in range(18):
                val = plsc.load_gather(stage.at[b], [a + v, rs, w])
                buf[b, r, pl.ds(v * 16, 16)] = val
            t2 = t + 12
            a2 = lax.shift_right_logical(t2, 4)
            w2 = jnp.bitwise_and(t2, 15)
            val = plsc.load_gather(stage.at[b], [a2 + 17, rs, w2])
            buf[b, r, pl.ds(284, 16)] = val
            return carry
        lax.fori_loop(0, CHUNK, body, 0)

    pend_g = {0: gathers(0, 0)}
    pend_w = {}
    for c in range(N_CHUNKS):
        b = c & 1
        for cp in pend_g.pop(c):
            cp.wait()
        if c + 1 < N_CHUNKS:
            pend_g[c + 1] = gathers(c + 1, (c + 1) & 1)
        if c >= 2:
            pend_w.pop(c - 2).wait()
        realign(c, b)
        pend_w[c] = pltpu.async_copy(
            buf.at[b], out_hbm.at[pl.ds(base + c * CHUNK, CHUNK)], wsem)
    for c in sorted(pend_w):
        pend_w[c].wait()


def kernel(pretrained, poses):
    tab16 = pretrained.reshape(NROWS16, SUBW)
    return _gather_kernel(tab16, poses.astype(jnp.int32))


# speedup vs baseline: 1.8610x; 1.8610x over previous
import functools

import jax
import jax.numpy as jnp
from jax import lax
from jax.experimental import pallas as pl
from jax.experimental.pallas import tpu as pltpu
from jax.experimental.pallas import tpu_sc as plsc

VOCAB = 100000
BATCH = 16384
DIM = 300
TAIL = DIM - 256

NC = 2
NS = 16
NW = NC * NS
CHUNK = 64
ROWS_PER_W = BATCH // NW
N_CHUNKS = ROWS_PER_W // CHUNK
SLOTS = 4

_mesh = plsc.VectorSubcoreMesh(core_axis_name="c", subcore_axis_name="s")


@functools.partial(
    pl.kernel,
    mesh=_mesh,
    out_type=jax.ShapeDtypeStruct((BATCH, DIM), jnp.float32),
    scratch_types=[
        pltpu.VMEM((ROWS_PER_W,), jnp.int32),
        pltpu.VMEM((SLOTS, 2, CHUNK, 128), jnp.float32),
        pltpu.SemaphoreType.DMA,
        pltpu.SemaphoreType.DMA,
        pltpu.SemaphoreType.DMA,
    ],
)
def _gather_kernel(tab, poses_hbm, out_hbm, idx_v, stage, gsem, wsem, tsem):
    wid = lax.axis_index("s") * NC + lax.axis_index("c")
    base = wid * ROWS_PER_W
    pltpu.sync_copy(poses_hbm.at[pl.ds(base, ROWS_PER_W)], idx_v)

    def gathers(c):
        s = c % SLOTS
        ids = idx_v.at[pl.ds(c * CHUNK, CHUNK)]
        return [
            pltpu.async_copy(tab.at[ids, pl.ds(0, 128)], stage.at[s, 0], gsem),
            pltpu.async_copy(tab.at[ids, pl.ds(128, 128)], stage.at[s, 1], gsem),
        ]

    def tails(c):
        def body(t, carry):
            vec = idx_v[pl.ds(c * CHUNK + t * 16, 16)]
            for lane in range(16):
                idx = vec[lane]
                pltpu.async_copy(
                    tab.at[idx, pl.ds(256, TAIL)],
                    out_hbm.at[base + c * CHUNK + t * 16 + lane,
                               pl.ds(256, TAIL)],
                    tsem,
                )
            return carry
        lax.fori_loop(0, CHUNK // 16, body, 0)

    def puts(c):
        s = c % SLOTS
        rows = pl.ds(base + c * CHUNK, CHUNK)
        return [
            pltpu.async_copy(stage.at[s, 0], out_hbm.at[rows, pl.ds(0, 128)], wsem),
            pltpu.async_copy(stage.at[s, 1], out_hbm.at[rows, pl.ds(128, 128)], wsem),
        ]

    pend_g = {c: gathers(c) for c in range(min(3, N_CHUNKS))}
    pend_w = {}
    for c in range(N_CHUNKS):
        tails(c)
        for cp in pend_g.pop(c):
            cp.wait()
        if c >= 1:
            for cp in pend_w.pop(c - 1):
                cp.wait()
        if c + 3 < N_CHUNKS:
            pend_g[c + 3] = gathers(c + 3)
        pend_w[c] = puts(c)
    pltpu.make_async_copy(
        tab.at[pl.ds(0, ROWS_PER_W), pl.ds(256, TAIL)],
        out_hbm.at[pl.ds(base, ROWS_PER_W), pl.ds(256, TAIL)],
        tsem,
    ).wait()
    for c in sorted(pend_w):
        for cp in pend_w[c]:
            cp.wait()


def kernel(pretrained, poses):
    return _gather_kernel(pretrained, poses.astype(jnp.int32))


# speedup vs baseline: 3.1547x; 1.6952x over previous
import functools

import jax
import jax.numpy as jnp
from jax import lax
from jax.experimental import pallas as pl
from jax.experimental.pallas import tpu as pltpu
from jax.experimental.pallas import tpu_sc as plsc

VOCAB = 100000
BATCH = 16384
DIM = 300
TAIL = DIM - 256

NC = 2
NS = 16
NW = NC * NS
CHUNK = 64
ROWS_PER_W = BATCH // NW
N_CHUNKS = ROWS_PER_W // CHUNK
SLOTS = 4

_mesh = plsc.VectorSubcoreMesh(core_axis_name="c", subcore_axis_name="s")


@functools.partial(
    pl.kernel,
    mesh=_mesh,
    out_type=jax.ShapeDtypeStruct((BATCH, 384), jnp.float32),
    scratch_types=[
        pltpu.VMEM((ROWS_PER_W,), jnp.int32),
        pltpu.VMEM((SLOTS, 3, CHUNK, 128), jnp.float32),
        pltpu.SemaphoreType.DMA,
        pltpu.SemaphoreType.DMA,
    ],
)
def _gather_kernel(tab, tail, poses_hbm, out_hbm, idx_v, stage, gsem, wsem):
    wid = lax.axis_index("s") * NC + lax.axis_index("c")
    base = wid * ROWS_PER_W
    pltpu.sync_copy(poses_hbm.at[pl.ds(base, ROWS_PER_W)], idx_v)

    def gather(c):
        s = c % SLOTS
        ids = idx_v.at[pl.ds(c * CHUNK, CHUNK)]
        return [
            pltpu.async_copy(tab.at[ids, pl.ds(0, 128)], stage.at[s, 0], gsem),
            pltpu.async_copy(tab.at[ids, pl.ds(128, 128)], stage.at[s, 1], gsem),
            pltpu.async_copy(tail.at[ids], stage.at[s, 2], gsem),
        ]

    def put(c):
        s = c % SLOTS
        rows = pl.ds(base + c * CHUNK, CHUNK)
        return [
            pltpu.async_copy(stage.at[s, 0], out_hbm.at[rows, pl.ds(0, 128)], wsem),
            pltpu.async_copy(stage.at[s, 1], out_hbm.at[rows, pl.ds(128, 128)], wsem),
            pltpu.async_copy(stage.at[s, 2], out_hbm.at[rows, pl.ds(256, 128)], wsem),
        ]

    pend_g = {c: gather(c) for c in range(min(3, N_CHUNKS))}
    pend_w = {}
    for c in range(N_CHUNKS):
        for cp in pend_g.pop(c):
            cp.wait()
        if c >= 1:
            for cp in pend_w.pop(c - 1):
                cp.wait()
        if c + 3 < N_CHUNKS:
            pend_g[c + 3] = gather(c + 3)
        pend_w[c] = put(c)
    for c in sorted(pend_w):
        for cp in pend_w[c]:
            cp.wait()


def kernel(pretrained, poses):
    tail = jnp.pad(lax.slice(pretrained, (0, 256), (VOCAB, DIM)),
                   ((0, 0), (0, 128 - TAIL)))
    out_pad = _gather_kernel(pretrained, tail, poses.astype(jnp.int32))
    return lax.slice(out_pad, (0, 0), (BATCH, DIM))


# speedup vs baseline: 3.5485x; 1.1248x over previous
import functools

import jax
import jax.numpy as jnp
from jax import lax
from jax.experimental import pallas as pl
from jax.experimental.pallas import tpu as pltpu
from jax.experimental.pallas import tpu_sc as plsc

VOCAB = 100000
BATCH = 16384
DIM = 300

NC = 2
NS = 16
NW = NC * NS
CHUNK = 64
ROWS_PER_W = BATCH // NW
N_CHUNKS = ROWS_PER_W // CHUNK
SLOTS = 4

TAIL_ROWS = 4000

_mesh = plsc.VectorSubcoreMesh(core_axis_name="c", subcore_axis_name="s")


def _tail_copy_kernel(x_ref, o_ref):
    o_ref[...] = x_ref[...]


def _build_tail(pretrained):
    return pl.pallas_call(
        _tail_copy_kernel,
        grid=(VOCAB // TAIL_ROWS,),
        in_specs=[pl.BlockSpec((TAIL_ROWS, 128), lambda i: (i, 2))],
        out_specs=pl.BlockSpec((TAIL_ROWS, 128), lambda i: (i, 0)),
        out_shape=jax.ShapeDtypeStruct((VOCAB, 128), jnp.float32),
    )(pretrained)


@functools.partial(
    pl.kernel,
    mesh=_mesh,
    out_type=jax.ShapeDtypeStruct((BATCH, 384), jnp.float32),
    scratch_types=[
        pltpu.VMEM((ROWS_PER_W,), jnp.int32),
        pltpu.VMEM((SLOTS, 3, CHUNK, 128), jnp.float32),
        pltpu.SemaphoreType.DMA,
        pltpu.SemaphoreType.DMA,
    ],
)
def _gather_kernel(tab, tail, poses_hbm, out_hbm, idx_v, stage, gsem, wsem):
    wid = lax.axis_index("s") * NC + lax.axis_index("c")
    base = wid * ROWS_PER_W
    pltpu.sync_copy(poses_hbm.at[pl.ds(base, ROWS_PER_W)], idx_v)

    def gather(c):
        s = c % SLOTS
        ids = idx_v.at[pl.ds(c * CHUNK, CHUNK)]
        return [
            pltpu.async_copy(tab.at[ids, pl.ds(0, 128)], stage.at[s, 0], gsem),
            pltpu.async_copy(tab.at[ids, pl.ds(128, 128)], stage.at[s, 1], gsem),
            pltpu.async_copy(tail.at[ids], stage.at[s, 2], gsem),
        ]

    def put(c):
        s = c % SLOTS
        rows = pl.ds(base + c * CHUNK, CHUNK)
        return [
            pltpu.async_copy(stage.at[s, 0], out_hbm.at[rows, pl.ds(0, 128)], wsem),
            pltpu.async_copy(stage.at[s, 1], out_hbm.at[rows, pl.ds(128, 128)], wsem),
            pltpu.async_copy(stage.at[s, 2], out_hbm.at[rows, pl.ds(256, 128)], wsem),
        ]

    pend_g = {c: gather(c) for c in range(min(3, N_CHUNKS))}
    pend_w = {}
    for c in range(N_CHUNKS):
        for cp in pend_g.pop(c):
            cp.wait()
        if c >= 1:
            for cp in pend_w.pop(c - 1):
                cp.wait()
        if c + 3 < N_CHUNKS:
            pend_g[c + 3] = gather(c + 3)
        pend_w[c] = put(c)
    for c in sorted(pend_w):
        for cp in pend_w[c]:
            cp.wait()


def kernel(pretrained, poses):
    tail = _build_tail(pretrained)
    out_pad = _gather_kernel(pretrained, tail, poses.astype(jnp.int32))
    return lax.slice(out_pad, (0, 0), (BATCH, DIM))


# speedup vs baseline: 3.5799x; 1.0088x over previous
import functools

import jax
import jax.numpy as jnp
from jax import lax
from jax.experimental import pallas as pl
from jax.experimental.pallas import tpu as pltpu
from jax.experimental.pallas import tpu_sc as plsc

VOCAB = 100000
BATCH = 16384
DIM = 300
PDIM = 384

NC = 2
NS = 16
NW = NC * NS
CHUNK = 64
ROWS_PER_W = BATCH // NW
N_CHUNKS = ROWS_PER_W // CHUNK
SLOTS = 3

TAIL_ROWS = 4000

_mesh = plsc.VectorSubcoreMesh(core_axis_name="c", subcore_axis_name="s")


def _tail_copy_kernel(x_ref, o_ref):
    o_ref[...] = x_ref[...]


def _build_tail(tab):
    return pl.pallas_call(
        _tail_copy_kernel,
        grid=(VOCAB // TAIL_ROWS,),
        in_specs=[pl.BlockSpec((TAIL_ROWS, 128), lambda i: (i, 2))],
        out_specs=pl.BlockSpec((TAIL_ROWS, 128), lambda i: (i, 0)),
        out_shape=jax.ShapeDtypeStruct((VOCAB, 128), jnp.float32),
    )(tab)


@functools.partial(
    pl.kernel,
    mesh=_mesh,
    out_type=jax.ShapeDtypeStruct((BATCH, PDIM), jnp.float32),
    scratch_types=[
        pltpu.VMEM((ROWS_PER_W,), jnp.int32),
        pltpu.VMEM((SLOTS, CHUNK, PDIM), jnp.float32),
        pltpu.SemaphoreType.DMA,
        pltpu.SemaphoreType.DMA,
    ],
)
def _gather_kernel(tab, tail, poses_hbm, out_hbm, idx_v, stage, gsem, wsem):
    wid = lax.axis_index("s") * NC + lax.axis_index("c")
    base = wid * ROWS_PER_W
    pltpu.sync_copy(poses_hbm.at[pl.ds(base, ROWS_PER_W)], idx_v)

    def gather(c):
        s = c % SLOTS
        ids = idx_v.at[pl.ds(c * CHUNK, CHUNK)]
        return [
            pltpu.async_copy(tab.at[ids, pl.ds(0, 256)],
                             stage.at[s, :, pl.ds(0, 256)], gsem),
            pltpu.async_copy(tail.at[ids],
                             stage.at[s, :, pl.ds(256, 128)], gsem),
        ]

    def put(c):
        rows = pl.ds(base + c * CHUNK, CHUNK)
        return pltpu.async_copy(stage.at[c % SLOTS], out_hbm.at[rows], wsem)

    pend_g = {c: gather(c) for c in range(min(SLOTS - 1, N_CHUNKS))}
    pend_w = {}
    for c in range(N_CHUNKS):
        for cp in pend_g.pop(c):
            cp.wait()
        if c >= 1:
            pend_w.pop(c - 1).wait()
        if c + SLOTS - 1 < N_CHUNKS:
            pend_g[c + SLOTS - 1] = gather(c + SLOTS - 1)
        pend_w[c] = put(c)
    for c in sorted(pend_w):
        pend_w[c].wait()


def kernel(pretrained, poses):
    tail = _build_tail(pretrained)
    out_pad = _gather_kernel(pretrained, tail, poses.astype(jnp.int32))
    return lax.slice(out_pad, (0, 0), (BATCH, DIM))
